# baseline (speedup 1.0000x reference)
import jax
import jax.numpy as jnp
from jax.experimental import pallas as pl
from jax.experimental.pallas import tpu as pltpu

_K = 32
_S = 250
_NDROP = 4


def _body(drop_ref, x_ref, o_ref):
    bd = pl.program_id(1)
    bh = pl.program_id(2)
    hits = []
    for n in range(_NDROP):
        p = drop_ref[n]
        hits.append((p // 64 == bd) & ((p // 8) % 8 == bh // 2))
    any_hit = hits[0] | hits[1] | hits[2] | hits[3]

    @pl.when(jnp.logical_not(any_hit))
    def _():
        o_ref[...] = x_ref[...]

    @pl.when(any_hit)
    def _():
        wp = jax.lax.broadcasted_iota(jnp.int32, (1, 1, 1, _S), 3) // _K
        mask = None
        for n in range(_NDROP):
            m = hits[n] & (drop_ref[n] % 8 == wp)
            mask = m if mask is None else mask | m
        o_ref[...] = jnp.where(mask, 0.0, x_ref[...])


def kernel(x, drop_idx):
    B = x.shape[0]
    xs = x.reshape(B, _S, _S, _S)
    nblk = (_S + _K - 1) // _K
    nh = (_S + 15) // 16
    out = pl.pallas_call(
        _body,
        grid=(B, nblk, nh),
        in_specs=[
            pl.BlockSpec(memory_space=pltpu.SMEM),
            pl.BlockSpec((1, _K, 16, _S), lambda b, i, j: (b, i, j, 0)),
        ],
        out_specs=pl.BlockSpec((1, _K, 16, _S), lambda b, i, j: (b, i, j, 0)),
        out_shape=jax.ShapeDtypeStruct((B, _S, _S, _S), jnp.float32),
        compiler_params=pltpu.CompilerParams(
            dimension_semantics=("parallel", "parallel", "parallel"),
        ),
    )(drop_idx.astype(jnp.int32), xs)
    return out.reshape(x.shape)
